# baseline (speedup 1.0000x reference)
import functools

import jax
import jax.numpy as jnp
from jax import lax
from jax.experimental import pallas as pl
from jax.experimental.pallas import tpu as pltpu
from jax.experimental.pallas import tpu_sc as plsc

NN = 10000
NP = 10240
EE = 320000
H = 128
NC, NS, L = 2, 16, 16
NW = NC * NS
EDGES_W = EE // NW
E2 = 327680
PR = E2 // 128
RW = PR // NW
AGR = 10112
VOCABS = (120, 10, 7, 5, 2)
EDIMS = (64, 16, 8, 8, 4)
NTASK = 12
F_ROWS = sum(VOCABS)
OFFS = (0, 120, 130, 137, 142)
BLK = 512

_mesh = plsc.VectorSubcoreMesh(core_axis_name="c", subcore_axis_name="s")
_sc_params = pltpu.CompilerParams(needs_layout_passes=False)
_sc_params_untiled = pltpu.CompilerParams(needs_layout_passes=False,
                                          use_tc_tiling_on_sc=False)


def _fold_body(a_ref, w_ref, b_ref, o_ref):
    o_ref[...] = jnp.dot(a_ref[...], w_ref[...],
                         preferred_element_type=jnp.float32) + b_ref[...]


def _fold(a, w, b):
    return pl.pallas_call(
        _fold_body,
        out_shape=jax.ShapeDtypeStruct(a.shape[:1] + (H,), jnp.float32),
    )(a, w, b)


def _embed_body(x_ref, fp_ref, t_ref, h_ref, r_ref):
    xb = x_ref[...]
    oh = jnp.zeros((BLK, 256), jnp.float32)
    for t in range(5):
        ci = jnp.clip(xb[:, t:t + 1], 0, VOCABS[t] - 1) + OFFS[t]
        oh = oh + (lax.broadcasted_iota(jnp.int32, (BLK, 256), 1)
                   == ci).astype(jnp.float32)
    h0 = jnp.dot(oh, fp_ref[...], preferred_element_type=jnp.float32)
    h_ref[...] = h0
    r_ref[...] = jnp.maximum(h0[:, None, :] + t_ref[...][None, :, :], 0.0)


def _embed(xpad, fp, t):
    nblk = NP // BLK
    return pl.pallas_call(
        _embed_body,
        grid=(nblk,),
        in_specs=[pl.BlockSpec((BLK, 8), lambda i: (i, 0)),
                  pl.BlockSpec((256, H), lambda i: (0, 0)),
                  pl.BlockSpec((16, H), lambda i: (0, 0))],
        out_specs=[pl.BlockSpec((BLK, H), lambda i: (i, 0)),
                   pl.BlockSpec((BLK, 16, H), lambda i: (i, 0, 0))],
        out_shape=[jax.ShapeDtypeStruct((NP, H), jnp.float32),
                   jax.ShapeDtypeStruct((NP, 16, H), jnp.float32)],
    )(xpad, fp, t)


def _prep_body(ei, ea, civ_hbm, dst_hbm, av2, sv, dvv, cv, sem):
    cid = lax.axis_index("c")
    sid = lax.axis_index("s")
    wid = cid * NS + sid
    iota = lax.iota(jnp.int32, L)
    iota3 = iota * 3
    col0 = jnp.zeros((L,), jnp.int32)

    def chunk(k, _):
        e0 = pl.multiple_of(wid * EDGES_W + 400 * k, 8)
        ca = pltpu.async_copy(ea.at[pl.ds(e0, 400)], av2, sem)
        cs = pltpu.async_copy(ei.at[pl.ds(e0, 400)], sv, sem)
        cd = pltpu.async_copy(ei.at[pl.ds(pl.multiple_of(EE + e0, 8), 400)],
                              dvv, sem)
        ca.wait()
        cs.wait()
        cd.wait()

        def grp(g, _):
            rows = iota + L * g
            a0 = plsc.load_gather(av2, [rows, col0])
            a1 = plsc.load_gather(av2, [rows, col0 + 1])
            a2 = plsc.load_gather(av2, [rows, col0 + 2])
            code = (a0 + 3 * a1 + 7 * a2) & 15
            cv[pl.ds(L * g, L)] = sv[pl.ds(L * g, L)] * 16 + code
            return 0

        lax.fori_loop(0, 400 // L, grp, 0)
        o1 = pltpu.async_copy(cv, civ_hbm.at[pl.ds(e0, 400)], sem)
        o2 = pltpu.async_copy(dvv, dst_hbm.at[pl.ds(e0, 400)], sem)
        o1.wait()
        o2.wait()
        return 0

    lax.fori_loop(0, EDGES_W // 400, chunk, 0)

    def pad16(g, _):
        cv[pl.ds(L * g, L)] = jnp.zeros((L,), jnp.int32)
        dvv[pl.ds(L * g, L)] = jnp.zeros((L,), jnp.int32) + NN
        return 0

    lax.fori_loop(0, 240 // L, pad16, 0)
    o1 = pltpu.async_copy(cv.at[pl.ds(0, 240)],
                          civ_hbm.at[pl.ds(pl.multiple_of(
                              EE + wid * 240, 8), 240)], sem)
    o2 = pltpu.async_copy(dvv.at[pl.ds(0, 240)],
                          dst_hbm.at[pl.ds(pl.multiple_of(
                              EE + wid * 240, 8), 240)], sem)
    o1.wait()
    o2.wait()


@functools.partial(
    pl.kernel,
    out_type=[jax.ShapeDtypeStruct((E2,), jnp.int32),
              jax.ShapeDtypeStruct((E2,), jnp.int32)],
    mesh=_mesh,
    compiler_params=_sc_params_untiled,
    scratch_types=(
        pltpu.VMEM((400, 3), jnp.int32),
        pltpu.VMEM((400,), jnp.int32),
        pltpu.VMEM((400,), jnp.int32),
        pltpu.VMEM((400,), jnp.int32),
        pltpu.SemaphoreType.DMA,
    ),
)
def _prep_kernel(ei, ea, *scratch):
    _prep_body(ei, ea, *scratch)


MC = 256


def _msg_body(r_hbm, civf, dstf, agg0, agg1, cif, dv2, hr, agg_sh,
              semi, sg0, sg1, sem_s):
    cid = lax.axis_index("c")
    sid = lax.axis_index("s")
    wid = cid * NS + sid
    sgs = (sg0, sg1)
    zero16 = jnp.zeros((L,), jnp.float32)

    def z(i, _):
        hr[i // (H // L), pl.ds((i % (H // L)) * L, L)] = zero16
        return 0

    lax.fori_loop(0, 128 * (H // L), z, 0)

    def zs(k, _):
        row = pl.multiple_of(128 * (sid + NS * k), 8)

        @pl.when(sid + NS * k < AGR // 128)
        def _():
            pltpu.sync_copy(hr.at[pl.ds(0, 128)],
                            agg_sh.at[pl.ds(row, 128)])

        return 0

    lax.fori_loop(0, (AGR // 128 + NS - 1) // NS, zs, 0)
    plsc.subcore_barrier()

    def chunk(k, _):
        e0 = pl.multiple_of(wid * (E2 // NW) + MC * k, 8)
        ics = [pltpu.async_copy(civf.at[pl.ds(e0, MC)], cif, semi)]
        for j in range(MC // 128):
            ics.append(pltpu.async_copy(
                dstf.at[pl.ds(pl.multiple_of(e0 + 128 * j, 8), 128)],
                dv2.at[j], semi))
        for c in ics:
            c.wait()
        gcps = [pltpu.async_copy(r_hbm.at[cif.at[pl.ds(128 * j, 128)]],
                                 hr.at[pl.ds(128 * j, 128)], sgs[j])
                for j in range(MC // 128)]
        scps = []
        for j in range(MC // 128):
            gcps[j].wait()
            scps.append(pltpu.async_copy(hr.at[pl.ds(128 * j, 128)],
                                         agg_sh.at[dv2.at[j]], sem_s,
                                         add=True))
        for sc in scps:
            sc.wait()
        return 0

    lax.fori_loop(0, E2 // NW // MC, chunk, 0)
    plsc.subcore_barrier()

    def out(k, _):
        r0 = pl.multiple_of(128 * (sid + NS * k), 8)

        @pl.when(sid + NS * k < AGR // 128)
        def _():
            @pl.when(cid == 0)
            def _():
                pltpu.sync_copy(agg_sh.at[pl.ds(r0, 128)],
                                agg0.at[pl.ds(r0, 128)])

            @pl.when(cid == 1)
            def _():
                pltpu.sync_copy(agg_sh.at[pl.ds(r0, 128)],
                                agg1.at[pl.ds(r0, 128)])

        return 0

    lax.fori_loop(0, (AGR // 128 + NS - 1) // NS, out, 0)


@functools.partial(
    pl.kernel,
    out_type=[jax.ShapeDtypeStruct((NP, H), jnp.float32),
              jax.ShapeDtypeStruct((NP, H), jnp.float32)],
    mesh=_mesh,
    compiler_params=_sc_params,
    scratch_types=(
        pltpu.VMEM((MC,), jnp.int32),
        pltpu.VMEM((MC // 128, 128), jnp.int32),
        pltpu.VMEM((MC, H), jnp.float32),
        pltpu.VMEM_SHARED((AGR, H), jnp.float32),
        pltpu.SemaphoreType.DMA,
        pltpu.SemaphoreType.DMA,
        pltpu.SemaphoreType.DMA,
        pltpu.SemaphoreType.DMA,
    ),
)
def _msg_kernel(r_hbm, civf, dstf, *scratch):
    _msg_body(r_hbm, civf, dstf, *scratch)


def _mlp_body(h_ref, a0_ref, a1_ref, epsp_ref, w1_ref, b1_ref, w2_ref,
              b2_ref, gs_ref, gb_ref, t_ref, o_ref, r_ref):
    u = h_ref[...] * epsp_ref[...] + a0_ref[...] + a1_ref[...]
    v = jnp.maximum(jnp.dot(u, w1_ref[...],
                            preferred_element_type=jnp.float32)
                    + b1_ref[...], 0.0)
    v = jnp.dot(v, w2_ref[...], preferred_element_type=jnp.float32) \
        + b2_ref[...]
    hn = jnp.maximum(v, 0.0) * gs_ref[...] + gb_ref[...]
    o_ref[...] = hn
    r_ref[...] = jnp.maximum(hn[:, None, :] + t_ref[...][None, :, :], 0.0)


def _mlp(h, a0, a1, epsp, w1, b1, w2, b2, gs, gb, t):
    grid = NP // BLK
    row_spec = pl.BlockSpec((BLK, H), lambda i: (i, 0))
    par_spec = pl.BlockSpec((H, H), lambda i: (0, 0))
    vec_spec = pl.BlockSpec((1, H), lambda i: (0, 0))
    return pl.pallas_call(
        _mlp_body,
        grid=(grid,),
        in_specs=[row_spec, row_spec, row_spec, vec_spec, par_spec, vec_spec,
                  par_spec, vec_spec, vec_spec, vec_spec,
                  pl.BlockSpec((16, H), lambda i: (0, 0))],
        out_specs=[row_spec, pl.BlockSpec((BLK, 16, H), lambda i: (i, 0, 0))],
        out_shape=[jax.ShapeDtypeStruct((NP, H), jnp.float32),
                   jax.ShapeDtypeStruct((NP, 16, H), jnp.float32)],
    )(h, a0, a1, epsp, w1, b1, w2, b2, gs, gb, t)


def _final_body(h_ref, a0_ref, a1_ref, epsp_ref, w1_ref, b1_ref, w2_ref,
                b2_ref, gw_ref, h1w_ref, h1b_ref, h2w_ref, h2b_ref, o_ref,
                m_ref, s_ref, acc_ref):
    i = pl.program_id(0)
    nblk = pl.num_programs(0)
    blk = h_ref.shape[0]

    @pl.when(i == 0)
    def _():
        m_ref[0] = -1e30
        s_ref[0] = 0.0
        acc_ref[...] = jnp.zeros_like(acc_ref)

    u = h_ref[...] * epsp_ref[...] + a0_ref[...] + a1_ref[...]
    v = jnp.maximum(jnp.dot(u, w1_ref[...],
                            preferred_element_type=jnp.float32)
                    + b1_ref[...], 0.0)
    h3 = jnp.maximum(jnp.dot(v, w2_ref[...],
                             preferred_element_type=jnp.float32)
                     + b2_ref[...], 0.0)
    rowid = i * blk + lax.broadcasted_iota(jnp.int32, (blk, 1), 0)
    valid = rowid < NN
    h3 = jnp.where(valid, h3, 0.0)
    g = jnp.sum(h3 * gw_ref[...], axis=1, keepdims=True)
    g = jnp.where(valid, g, -1e30)
    mo = m_ref[0]
    mn = jnp.maximum(mo, jnp.max(g))
    corr = jnp.exp(mo - mn)
    e = jnp.where(valid, jnp.exp(g - mn), 0.0)
    s_ref[0] = s_ref[0] * corr + jnp.sum(e)
    acc_ref[...] = acc_ref[...] * corr + jnp.sum(e * h3, axis=0,
                                                 keepdims=True)
    m_ref[0] = mn

    @pl.when(i == nblk - 1)
    def _():
        hg = acc_ref[...] / s_ref[0]
        z = jnp.maximum(jnp.dot(hg, h1w_ref[...],
                                preferred_element_type=jnp.float32)
                        + h1b_ref[...], 0.0)
        lg = jnp.dot(z, h2w_ref[...],
                     preferred_element_type=jnp.float32) + h2b_ref[...]
        o_ref[...] = jnp.broadcast_to(lg, o_ref.shape)


def _final(h, a0, a1, epsp, w1, b1, w2, b2, gw, h1w, h1b, h2w, h2b):
    grid = NP // BLK
    row_spec = pl.BlockSpec((BLK, H), lambda i: (i, 0))
    par_spec = pl.BlockSpec((H, H), lambda i: (0, 0))
    vec_spec = pl.BlockSpec((1, H), lambda i: (0, 0))
    return pl.pallas_call(
        _final_body,
        grid=(grid,),
        in_specs=[row_spec, row_spec, row_spec, vec_spec, par_spec, vec_spec,
                  par_spec, vec_spec, vec_spec, par_spec, vec_spec, par_spec,
                  vec_spec],
        out_specs=pl.BlockSpec((8, H), lambda i: (0, 0)),
        out_shape=jax.ShapeDtypeStruct((8, H), jnp.float32),
        scratch_shapes=[pltpu.SMEM((1,), jnp.float32),
                        pltpu.SMEM((1,), jnp.float32),
                        pltpu.VMEM((1, H), jnp.float32)],
    )(h, a0, a1, epsp, w1, b1, w2, b2, gw, h1w, h1b, h2w, h2b)


def kernel(x, edge_index, edge_attr, batch, params):
    f32 = jnp.float32
    A = jnp.zeros((192, 128), f32)
    col = 0
    for t in range(5):
        A = A.at[OFFS[t]:OFFS[t] + VOCABS[t], col:col + EDIMS[t]].set(
            params['node_embs'][t])
        col += EDIMS[t]
    for l, name in enumerate(('g1', 'g2', 'g3')):
        A = A.at[F_ROWS + 16 * l:F_ROWS + 16 * (l + 1),
                 100 + 8 * l:100 + 8 * (l + 1)].set(params[name]['edge_emb'])
    W = jnp.zeros((128, H), f32)
    W = W.at[0:100].set(params['proj_w'])
    for l, name in enumerate(('g1', 'g2', 'g3')):
        W = W.at[100 + 8 * l:100 + 8 * (l + 1)].set(params[name]['lin_w'])
    B = jnp.zeros((192, H), f32)
    B = B.at[0:F_ROWS].set(params['proj_b'][None, :] / 5.0)
    for l, name in enumerate(('g1', 'g2', 'g3')):
        B = B.at[F_ROWS + 16 * l:F_ROWS + 16 * (l + 1)].set(
            params[name]['lin_b'][None, :])

    FT = _fold(A, W, B)
    Fp = jnp.pad(FT, ((0, 64), (0, 0)))
    T = [FT[F_ROWS + 16 * l:F_ROWS + 16 * (l + 1)] for l in range(3)]

    xpad = jnp.pad(x, ((0, NP - NN), (0, 3)))
    ei = edge_index.reshape(-1)
    civf, dstf = _prep_kernel(ei, edge_attr)

    h, R = _embed(xpad, Fp, T[0])

    bn_scale = 1.0 / jnp.sqrt(1.0 + 1e-5)
    ones = jnp.ones((1, H), f32)
    for l, name in enumerate(('g1', 'g2', 'g3')):
        bp = params[name]
        agg = _msg_kernel(R.reshape(NP * 16, H), civf, dstf)
        epsp = (1.0 + bp['eps']) * ones
        b1 = bp['b1'][None, :]
        b2 = bp['b2'][None, :]
        if l < 2:
            gs = (params[f'bn{l + 1}_g'] * bn_scale)[None, :]
            gb = params[f'bn{l + 1}_b'][None, :]
            h, R = _mlp(h, agg[0], agg[1], epsp, bp['w1'], b1, bp['w2'],
                        b2, gs, gb, T[l + 1])
        else:
            gw = params['gate_w'].reshape(1, H)
            h2w = jnp.zeros((H, H), f32).at[:, :NTASK].set(params['h2_w'])
            h2b = jnp.zeros((1, H), f32).at[0, :NTASK].set(params['h2_b'])
            out = _final(h, agg[0], agg[1], epsp, bp['w1'], b1, bp['w2'],
                         b2, gw, params['h1_w'], params['h1_b'][None, :],
                         h2w, h2b)
    return out[0:1, 0:NTASK]


# speedup vs baseline: 2.2356x; 2.2356x over previous
import functools

import jax
import jax.numpy as jnp
from jax import lax
from jax.experimental import pallas as pl
from jax.experimental.pallas import tpu as pltpu
from jax.experimental.pallas import tpu_sc as plsc

NN = 10000
NP = 10240
EE = 320000
H = 128
NC, NS, L = 2, 16, 16
NW = NC * NS
EDGES_W = EE // NW
E2 = 327680
PR = E2 // 128
RW = PR // NW
AGR = 10112
VOCABS = (120, 10, 7, 5, 2)
EDIMS = (64, 16, 8, 8, 4)
NTASK = 12
F_ROWS = sum(VOCABS)
OFFS = (0, 120, 130, 137, 142)
BLK = 512

_mesh = plsc.VectorSubcoreMesh(core_axis_name="c", subcore_axis_name="s")
_sc_params = pltpu.CompilerParams(needs_layout_passes=False)
_sc_params_untiled = pltpu.CompilerParams(needs_layout_passes=False,
                                          use_tc_tiling_on_sc=False)


def _fold_body(a_ref, w_ref, b_ref, o_ref):
    o_ref[...] = jnp.dot(a_ref[...], w_ref[...],
                         preferred_element_type=jnp.float32) + b_ref[...]


def _fold(a, w, b):
    return pl.pallas_call(
        _fold_body,
        out_shape=jax.ShapeDtypeStruct(a.shape[:1] + (H,), jnp.float32),
    )(a, w, b)


def _embed_body(x_ref, fp_ref, t_ref, h_ref, r_ref):
    xb = x_ref[...]
    oh = jnp.zeros((BLK, 256), jnp.float32)
    for t in range(5):
        ci = jnp.clip(xb[:, t:t + 1], 0, VOCABS[t] - 1) + OFFS[t]
        oh = oh + (lax.broadcasted_iota(jnp.int32, (BLK, 256), 1)
                   == ci).astype(jnp.float32)
    h0 = jnp.dot(oh, fp_ref[...], preferred_element_type=jnp.float32)
    h_ref[...] = h0
    r_ref[...] = jnp.maximum(h0[:, None, :] + t_ref[...][None, :, :], 0.0)


def _embed(xpad, fp, t):
    nblk = NP // BLK
    return pl.pallas_call(
        _embed_body,
        grid=(nblk,),
        in_specs=[pl.BlockSpec((BLK, 8), lambda i: (i, 0)),
                  pl.BlockSpec((256, H), lambda i: (0, 0)),
                  pl.BlockSpec((16, H), lambda i: (0, 0))],
        out_specs=[pl.BlockSpec((BLK, H), lambda i: (i, 0)),
                   pl.BlockSpec((BLK, 16, H), lambda i: (i, 0, 0))],
        out_shape=[jax.ShapeDtypeStruct((NP, H), jnp.float32),
                   jax.ShapeDtypeStruct((NP, 16, H), jnp.float32)],
    )(xpad, fp, t)


def _prep_body(ei, ea, civ_hbm, av2, sv, cv, sem):
    cid = lax.axis_index("c")
    sid = lax.axis_index("s")
    wid = cid * NS + sid
    iota = lax.iota(jnp.int32, L)
    col0 = jnp.zeros((L,), jnp.int32)

    def chunk(k, _):
        e0 = pl.multiple_of(wid * EDGES_W + 400 * k, 8)
        ca = pltpu.async_copy(ea.at[pl.ds(e0, 400)], av2, sem)
        cs = pltpu.async_copy(ei.at[pl.ds(e0, 400)], sv, sem)
        ca.wait()
        cs.wait()

        def grp(g, _):
            rows = iota + L * g
            a0 = plsc.load_gather(av2, [rows, col0])
            a1 = plsc.load_gather(av2, [rows, col0 + 1])
            a2 = plsc.load_gather(av2, [rows, col0 + 2])
            code = (a0 + 3 * a1 + 7 * a2) & 15
            cv[pl.ds(L * g, L)] = sv[pl.ds(L * g, L)] * 16 + code
            return 0

        lax.fori_loop(0, 400 // L, grp, 0)
        pltpu.sync_copy(cv, civ_hbm.at[pl.ds(e0, 400)])
        return 0

    lax.fori_loop(0, EDGES_W // 400, chunk, 0)


@functools.partial(
    pl.kernel,
    out_type=jax.ShapeDtypeStruct((EE,), jnp.int32),
    mesh=_mesh,
    compiler_params=_sc_params_untiled,
    scratch_types=(
        pltpu.VMEM((400, 3), jnp.int32),
        pltpu.VMEM((400,), jnp.int32),
        pltpu.VMEM((400,), jnp.int32),
        pltpu.SemaphoreType.DMA,
    ),
)
def _prep_kernel(ei, ea, *scratch):
    _prep_body(ei, ea, *scratch)


CH = 320
SUB = 80


def _msg_body(r_hbm, civ, ei, agg0, agg1, civ2, dv2, hr, agg_sh,
              semi, sg0, sg1, sg2, sg3, sem_s):
    cid = lax.axis_index("c")
    sid = lax.axis_index("s")
    wid = cid * NS + sid
    sgs = (sg0, sg1, sg2, sg3)
    zero16 = jnp.zeros((L,), jnp.float32)

    def z(i, _):
        hr[i // (H // L), pl.ds((i % (H // L)) * L, L)] = zero16
        return 0

    lax.fori_loop(0, SUB * (H // L), z, 0)

    def zs(k, _):
        pltpu.sync_copy(hr.at[pl.ds(0, SUB)],
                        agg_sh.at[pl.ds(pl.multiple_of(
                            sid * (NP // NS) + SUB * k, 8), SUB)])
        return 0

    lax.fori_loop(0, NP // NS // SUB, zs, 0)
    plsc.subcore_barrier()

    def do_chunk(e0, n_sub):
        ics = []
        for j in range(n_sub):
            o = pl.multiple_of(e0 + SUB * j, 8)
            ics.append(pltpu.async_copy(civ.at[pl.ds(o, SUB)],
                                        civ2.at[j], semi))
            ics.append(pltpu.async_copy(
                ei.at[pl.ds(pl.multiple_of(EE + e0 + SUB * j, 8), SUB)],
                dv2.at[j], semi))
        for c in ics:
            c.wait()
        gcps = [pltpu.async_copy(r_hbm.at[civ2.at[j]],
                                 hr.at[pl.ds(SUB * j, SUB)], sgs[j])
                for j in range(n_sub)]
        scps = []
        for j in range(n_sub):
            gcps[j].wait()
            scps.append(pltpu.async_copy(hr.at[pl.ds(SUB * j, SUB)],
                                         agg_sh.at[dv2.at[j]], sem_s,
                                         add=True))
        for sc in scps:
            sc.wait()

    def chunk(k, _):
        do_chunk(wid * EDGES_W + CH * k, CH // SUB)
        return 0

    nfull = EDGES_W // CH
    lax.fori_loop(0, nfull, chunk, 0)
    do_chunk(wid * EDGES_W + CH * nfull, (EDGES_W - CH * nfull) // SUB)
    plsc.subcore_barrier()

    def out(k, _):
        r0 = pl.multiple_of(sid * (NP // NS) + SUB * k, 8)

        @pl.when(cid == 0)
        def _():
            pltpu.sync_copy(agg_sh.at[pl.ds(r0, SUB)],
                            agg0.at[pl.ds(r0, SUB)])

        @pl.when(cid == 1)
        def _():
            pltpu.sync_copy(agg_sh.at[pl.ds(r0, SUB)],
                            agg1.at[pl.ds(r0, SUB)])

        return 0

    lax.fori_loop(0, NP // NS // SUB, out, 0)


@functools.partial(
    pl.kernel,
    out_type=[jax.ShapeDtypeStruct((NP, H), jnp.float32),
              jax.ShapeDtypeStruct((NP, H), jnp.float32)],
    mesh=_mesh,
    compiler_params=_sc_params,
    scratch_types=(
        pltpu.VMEM((CH // SUB, SUB), jnp.int32),
        pltpu.VMEM((CH // SUB, SUB), jnp.int32),
        pltpu.VMEM((CH, H), jnp.float32),
        pltpu.VMEM_SHARED((NP, H), jnp.float32),
        pltpu.SemaphoreType.DMA,
        pltpu.SemaphoreType.DMA,
        pltpu.SemaphoreType.DMA,
        pltpu.SemaphoreType.DMA,
        pltpu.SemaphoreType.DMA,
        pltpu.SemaphoreType.DMA,
    ),
)
def _msg_kernel(r_hbm, civ, ei, *scratch):
    _msg_body(r_hbm, civ, ei, *scratch)


def _mlp_body(h_ref, a0_ref, a1_ref, epsp_ref, w1_ref, b1_ref, w2_ref,
              b2_ref, gs_ref, gb_ref, t_ref, o_ref, r_ref):
    u = h_ref[...] * epsp_ref[...] + a0_ref[...] + a1_ref[...]
    v = jnp.maximum(jnp.dot(u, w1_ref[...],
                            preferred_element_type=jnp.float32)
                    + b1_ref[...], 0.0)
    v = jnp.dot(v, w2_ref[...], preferred_element_type=jnp.float32) \
        + b2_ref[...]
    hn = jnp.maximum(v, 0.0) * gs_ref[...] + gb_ref[...]
    o_ref[...] = hn
    r_ref[...] = jnp.maximum(hn[:, None, :] + t_ref[...][None, :, :], 0.0)


def _mlp(h, a0, a1, epsp, w1, b1, w2, b2, gs, gb, t):
    grid = NP // BLK
    row_spec = pl.BlockSpec((BLK, H), lambda i: (i, 0))
    par_spec = pl.BlockSpec((H, H), lambda i: (0, 0))
    vec_spec = pl.BlockSpec((1, H), lambda i: (0, 0))
    return pl.pallas_call(
        _mlp_body,
        grid=(grid,),
        in_specs=[row_spec, row_spec, row_spec, vec_spec, par_spec, vec_spec,
                  par_spec, vec_spec, vec_spec, vec_spec,
                  pl.BlockSpec((16, H), lambda i: (0, 0))],
        out_specs=[row_spec, pl.BlockSpec((BLK, 16, H), lambda i: (i, 0, 0))],
        out_shape=[jax.ShapeDtypeStruct((NP, H), jnp.float32),
                   jax.ShapeDtypeStruct((NP, 16, H), jnp.float32)],
    )(h, a0, a1, epsp, w1, b1, w2, b2, gs, gb, t)


def _final_body(h_ref, a0_ref, a1_ref, epsp_ref, w1_ref, b1_ref, w2_ref,
                b2_ref, gw_ref, h1w_ref, h1b_ref, h2w_ref, h2b_ref, o_ref,
                m_ref, s_ref, acc_ref):
    i = pl.program_id(0)
    nblk = pl.num_programs(0)
    blk = h_ref.shape[0]

    @pl.when(i == 0)
    def _():
        m_ref[0] = -1e30
        s_ref[0] = 0.0
        acc_ref[...] = jnp.zeros_like(acc_ref)

    u = h_ref[...] * epsp_ref[...] + a0_ref[...] + a1_ref[...]
    v = jnp.maximum(jnp.dot(u, w1_ref[...],
                            preferred_element_type=jnp.float32)
                    + b1_ref[...], 0.0)
    h3 = jnp.maximum(jnp.dot(v, w2_ref[...],
                             preferred_element_type=jnp.float32)
                     + b2_ref[...], 0.0)
    rowid = i * blk + lax.broadcasted_iota(jnp.int32, (blk, 1), 0)
    valid = rowid < NN
    h3 = jnp.where(valid, h3, 0.0)
    g = jnp.sum(h3 * gw_ref[...], axis=1, keepdims=True)
    g = jnp.where(valid, g, -1e30)
    mo = m_ref[0]
    mn = jnp.maximum(mo, jnp.max(g))
    corr = jnp.exp(mo - mn)
    e = jnp.where(valid, jnp.exp(g - mn), 0.0)
    s_ref[0] = s_ref[0] * corr + jnp.sum(e)
    acc_ref[...] = acc_ref[...] * corr + jnp.sum(e * h3, axis=0,
                                                 keepdims=True)
    m_ref[0] = mn

    @pl.when(i == nblk - 1)
    def _():
        hg = acc_ref[...] / s_ref[0]
        z = jnp.maximum(jnp.dot(hg, h1w_ref[...],
                                preferred_element_type=jnp.float32)
                        + h1b_ref[...], 0.0)
        lg = jnp.dot(z, h2w_ref[...],
                     preferred_element_type=jnp.float32) + h2b_ref[...]
        o_ref[...] = jnp.broadcast_to(lg, o_ref.shape)


def _final(h, a0, a1, epsp, w1, b1, w2, b2, gw, h1w, h1b, h2w, h2b):
    grid = NP // BLK
    row_spec = pl.BlockSpec((BLK, H), lambda i: (i, 0))
    par_spec = pl.BlockSpec((H, H), lambda i: (0, 0))
    vec_spec = pl.BlockSpec((1, H), lambda i: (0, 0))
    return pl.pallas_call(
        _final_body,
        grid=(grid,),
        in_specs=[row_spec, row_spec, row_spec, vec_spec, par_spec, vec_spec,
                  par_spec, vec_spec, vec_spec, par_spec, vec_spec, par_spec,
                  vec_spec],
        out_specs=pl.BlockSpec((8, H), lambda i: (0, 0)),
        out_shape=jax.ShapeDtypeStruct((8, H), jnp.float32),
        scratch_shapes=[pltpu.SMEM((1,), jnp.float32),
                        pltpu.SMEM((1,), jnp.float32),
                        pltpu.VMEM((1, H), jnp.float32)],
    )(h, a0, a1, epsp, w1, b1, w2, b2, gw, h1w, h1b, h2w, h2b)


def kernel(x, edge_index, edge_attr, batch, params):
    f32 = jnp.float32
    A = jnp.zeros((192, 128), f32)
    col = 0
    for t in range(5):
        A = A.at[OFFS[t]:OFFS[t] + VOCABS[t], col:col + EDIMS[t]].set(
            params['node_embs'][t])
        col += EDIMS[t]
    for l, name in enumerate(('g1', 'g2', 'g3')):
        A = A.at[F_ROWS + 16 * l:F_ROWS + 16 * (l + 1),
                 100 + 8 * l:100 + 8 * (l + 1)].set(params[name]['edge_emb'])
    W = jnp.zeros((128, H), f32)
    W = W.at[0:100].set(params['proj_w'])
    for l, name in enumerate(('g1', 'g2', 'g3')):
        W = W.at[100 + 8 * l:100 + 8 * (l + 1)].set(params[name]['lin_w'])
    B = jnp.zeros((192, H), f32)
    B = B.at[0:F_ROWS].set(params['proj_b'][None, :] / 5.0)
    for l, name in enumerate(('g1', 'g2', 'g3')):
        B = B.at[F_ROWS + 16 * l:F_ROWS + 16 * (l + 1)].set(
            params[name]['lin_b'][None, :])

    FT = _fold(A, W, B)
    Fp = jnp.pad(FT, ((0, 64), (0, 0)))
    T = [FT[F_ROWS + 16 * l:F_ROWS + 16 * (l + 1)] for l in range(3)]

    xpad = jnp.pad(x, ((0, NP - NN), (0, 3)))
    ei = edge_index.reshape(-1)
    civ = _prep_kernel(ei, edge_attr)

    h, R = _embed(xpad, Fp, T[0])

    bn_scale = 1.0 / jnp.sqrt(1.0 + 1e-5)
    ones = jnp.ones((1, H), f32)
    for l, name in enumerate(('g1', 'g2', 'g3')):
        bp = params[name]
        agg = _msg_kernel(R.reshape(NP * 16, H), civ, ei)
        epsp = (1.0 + bp['eps']) * ones
        b1 = bp['b1'][None, :]
        b2 = bp['b2'][None, :]
        if l < 2:
            gs = (params[f'bn{l + 1}_g'] * bn_scale)[None, :]
            gb = params[f'bn{l + 1}_b'][None, :]
            h, R = _mlp(h, agg[0], agg[1], epsp, bp['w1'], b1, bp['w2'],
                        b2, gs, gb, T[l + 1])
        else:
            gw = params['gate_w'].reshape(1, H)
            h2w = jnp.zeros((H, H), f32).at[:, :NTASK].set(params['h2_w'])
            h2b = jnp.zeros((1, H), f32).at[0, :NTASK].set(params['h2_b'])
            out = _final(h, agg[0], agg[1], epsp, bp['w1'], b1, bp['w2'],
                         b2, gw, params['h1_w'], params['h1_b'][None, :],
                         h2w, h2b)
    return out[0:1, 0:NTASK]


# speedup vs baseline: 2.6938x; 1.2050x over previous
import functools

import jax
import jax.numpy as jnp
from jax import lax
from jax.experimental import pallas as pl
from jax.experimental.pallas import tpu as pltpu
from jax.experimental.pallas import tpu_sc as plsc

NN = 10000
NP = 10240
EE = 320000
H = 128
NC, NS, L = 2, 16, 16
NW = NC * NS
EDGES_W = EE // NW
E2 = 327680
PR = E2 // 128
RW = PR // NW
AGR = 10112
VOCABS = (120, 10, 7, 5, 2)
EDIMS = (64, 16, 8, 8, 4)
NTASK = 12
F_ROWS = sum(VOCABS)
OFFS = (0, 120, 130, 137, 142)
BLK = 512

_mesh = plsc.VectorSubcoreMesh(core_axis_name="c", subcore_axis_name="s")
_sc_params = pltpu.CompilerParams(needs_layout_passes=False)
_sc_params_untiled = pltpu.CompilerParams(needs_layout_passes=False,
                                          use_tc_tiling_on_sc=False)


def _fold_body(a_ref, w_ref, b_ref, o_ref):
    o_ref[...] = jnp.dot(a_ref[...], w_ref[...],
                         preferred_element_type=jnp.float32) + b_ref[...]


def _fold(a, w, b):
    return pl.pallas_call(
        _fold_body,
        out_shape=jax.ShapeDtypeStruct(a.shape[:1] + (H,), jnp.float32),
    )(a, w, b)


def _embed_body(x_ref, fp_ref, t_ref, h_ref, r_ref):
    xb = x_ref[...]
    oh = jnp.zeros((BLK, 256), jnp.float32)
    for t in range(5):
        ci = jnp.clip(xb[:, t:t + 1], 0, VOCABS[t] - 1) + OFFS[t]
        oh = oh + (lax.broadcasted_iota(jnp.int32, (BLK, 256), 1)
                   == ci).astype(jnp.float32)
    h0 = jnp.dot(oh, fp_ref[...], preferred_element_type=jnp.float32)
    h_ref[...] = h0
    r_ref[...] = jnp.maximum(h0[:, None, :] + t_ref[...][None, :, :], 0.0)


def _embed(xpad, fp, t):
    nblk = NP // BLK
    return pl.pallas_call(
        _embed_body,
        grid=(nblk,),
        in_specs=[pl.BlockSpec((BLK, 8), lambda i: (i, 0)),
                  pl.BlockSpec((256, H), lambda i: (0, 0)),
                  pl.BlockSpec((16, H), lambda i: (0, 0))],
        out_specs=[pl.BlockSpec((BLK, H), lambda i: (i, 0)),
                   pl.BlockSpec((BLK, 16, H), lambda i: (i, 0, 0))],
        out_shape=[jax.ShapeDtypeStruct((NP, H), jnp.float32),
                   jax.ShapeDtypeStruct((NP, 16, H), jnp.float32)],
    )(xpad, fp, t)


def _prep_body(ei, ea, civ_hbm, av2, sv, cv, sem):
    cid = lax.axis_index("c")
    sid = lax.axis_index("s")
    wid = cid * NS + sid
    iota = lax.iota(jnp.int32, L)
    col0 = jnp.zeros((L,), jnp.int32)

    def chunk(k, _):
        e0 = pl.multiple_of(wid * EDGES_W + 400 * k, 8)
        ca = pltpu.async_copy(ea.at[pl.ds(e0, 400)], av2, sem)
        cs = pltpu.async_copy(ei.at[pl.ds(e0, 400)], sv, sem)
        ca.wait()
        cs.wait()

        def grp(g, _):
            rows = iota + L * g
            a0 = plsc.load_gather(av2, [rows, col0])
            a1 = plsc.load_gather(av2, [rows, col0 + 1])
            a2 = plsc.load_gather(av2, [rows, col0 + 2])
            code = (a0 + 3 * a1 + 7 * a2) & 15
            cv[pl.ds(L * g, L)] = sv[pl.ds(L * g, L)] * 16 + code
            return 0

        lax.fori_loop(0, 400 // L, grp, 0)
        pltpu.sync_copy(cv, civ_hbm.at[pl.ds(e0, 400)])
        return 0

    lax.fori_loop(0, EDGES_W // 400, chunk, 0)


@functools.partial(
    pl.kernel,
    out_type=jax.ShapeDtypeStruct((EE,), jnp.int32),
    mesh=_mesh,
    compiler_params=_sc_params,
    scratch_types=(
        pltpu.VMEM((400, 3), jnp.int32),
        pltpu.VMEM((400,), jnp.int32),
        pltpu.VMEM((400,), jnp.int32),
        pltpu.SemaphoreType.DMA,
    ),
)
def _prep_kernel(ei, ea, *scratch):
    _prep_body(ei, ea, *scratch)


CH = 320
SUB = 80


def _msg_body(r_hbm, civ, ei, agg0, agg1, civ2, dv2, hr, agg_sh,
              semi, sg0, sg1, sg2, sg3, sem_s):
    cid = lax.axis_index("c")
    sid = lax.axis_index("s")
    wid = cid * NS + sid
    sgs = (sg0, sg1, sg2, sg3)
    zero16 = jnp.zeros((L,), jnp.float32)

    def z(i, _):
        hr[i // (H // L), pl.ds((i % (H // L)) * L, L)] = zero16
        return 0

    lax.fori_loop(0, SUB * (H // L), z, 0)

    def zs(k, _):
        pltpu.sync_copy(hr.at[pl.ds(0, SUB)],
                        agg_sh.at[pl.ds(pl.multiple_of(
                            sid * (NP // NS) + SUB * k, 8), SUB)])
        return 0

    lax.fori_loop(0, NP // NS // SUB, zs, 0)
    plsc.subcore_barrier()

    def do_chunk(e0, n_sub):
        ics = []
        for j in range(n_sub):
            o = pl.multiple_of(e0 + SUB * j, 8)
            ics.append(pltpu.async_copy(civ.at[pl.ds(o, SUB)],
                                        civ2.at[j], semi))
            ics.append(pltpu.async_copy(
                ei.at[pl.ds(pl.multiple_of(EE + e0 + SUB * j, 8), SUB)],
                dv2.at[j], semi))
        for c in ics:
            c.wait()
        gcps = [pltpu.async_copy(r_hbm.at[civ2.at[j]],
                                 hr.at[pl.ds(SUB * j, SUB)], sgs[j])
                for j in range(n_sub)]
        scps = []
        for j in range(n_sub):
            gcps[j].wait()
            scps.append(pltpu.async_copy(hr.at[pl.ds(SUB * j, SUB)],
                                         agg_sh.at[dv2.at[j]], sem_s,
                                         add=True))
        for sc in scps:
            sc.wait()

    def chunk(k, _):
        do_chunk(wid * EDGES_W + CH * k, CH // SUB)
        return 0

    nfull = EDGES_W // CH
    lax.fori_loop(0, nfull, chunk, 0)
    do_chunk(wid * EDGES_W + CH * nfull, (EDGES_W - CH * nfull) // SUB)
    plsc.subcore_barrier()

    def out(k, _):
        r0 = pl.multiple_of(sid * (NP // NS) + SUB * k, 8)

        @pl.when(cid == 0)
        def _():
            pltpu.sync_copy(agg_sh.at[pl.ds(r0, SUB)],
                            agg0.at[pl.ds(r0, SUB)])

        @pl.when(cid == 1)
        def _():
            pltpu.sync_copy(agg_sh.at[pl.ds(r0, SUB)],
                            agg1.at[pl.ds(r0, SUB)])

        return 0

    lax.fori_loop(0, NP // NS // SUB, out, 0)


@functools.partial(
    pl.kernel,
    out_type=[jax.ShapeDtypeStruct((NP, H), jnp.float32),
              jax.ShapeDtypeStruct((NP, H), jnp.float32)],
    mesh=_mesh,
    compiler_params=_sc_params,
    scratch_types=(
        pltpu.VMEM((CH // SUB, SUB), jnp.int32),
        pltpu.VMEM((CH // SUB, SUB), jnp.int32),
        pltpu.VMEM((CH, H), jnp.float32),
        pltpu.VMEM_SHARED((NP, H), jnp.float32),
        pltpu.SemaphoreType.DMA,
        pltpu.SemaphoreType.DMA,
        pltpu.SemaphoreType.DMA,
        pltpu.SemaphoreType.DMA,
        pltpu.SemaphoreType.DMA,
        pltpu.SemaphoreType.DMA,
    ),
)
def _msg_kernel(r_hbm, civ, ei, *scratch):
    _msg_body(r_hbm, civ, ei, *scratch)


def _mlp_body(h_ref, a0_ref, a1_ref, epsp_ref, w1_ref, b1_ref, w2_ref,
              b2_ref, gs_ref, gb_ref, t_ref, o_ref, r_ref):
    u = h_ref[...] * epsp_ref[...] + a0_ref[...] + a1_ref[...]
    v = jnp.maximum(jnp.dot(u, w1_ref[...],
                            preferred_element_type=jnp.float32)
                    + b1_ref[...], 0.0)
    v = jnp.dot(v, w2_ref[...], preferred_element_type=jnp.float32) \
        + b2_ref[...]
    hn = jnp.maximum(v, 0.0) * gs_ref[...] + gb_ref[...]
    o_ref[...] = hn
    r_ref[...] = jnp.maximum(hn[:, None, :] + t_ref[...][None, :, :], 0.0)


def _mlp(h, a0, a1, epsp, w1, b1, w2, b2, gs, gb, t):
    grid = NP // BLK
    row_spec = pl.BlockSpec((BLK, H), lambda i: (i, 0))
    par_spec = pl.BlockSpec((H, H), lambda i: (0, 0))
    vec_spec = pl.BlockSpec((1, H), lambda i: (0, 0))
    return pl.pallas_call(
        _mlp_body,
        grid=(grid,),
        in_specs=[row_spec, row_spec, row_spec, vec_spec, par_spec, vec_spec,
                  par_spec, vec_spec, vec_spec, vec_spec,
                  pl.BlockSpec((16, H), lambda i: (0, 0))],
        out_specs=[row_spec, pl.BlockSpec((BLK, 16, H), lambda i: (i, 0, 0))],
        out_shape=[jax.ShapeDtypeStruct((NP, H), jnp.float32),
                   jax.ShapeDtypeStruct((NP, 16, H), jnp.float32)],
    )(h, a0, a1, epsp, w1, b1, w2, b2, gs, gb, t)


def _final_body(h_ref, a0_ref, a1_ref, epsp_ref, w1_ref, b1_ref, w2_ref,
                b2_ref, gw_ref, h1w_ref, h1b_ref, h2w_ref, h2b_ref, o_ref,
                m_ref, s_ref, acc_ref):
    i = pl.program_id(0)
    nblk = pl.num_programs(0)
    blk = h_ref.shape[0]

    @pl.when(i == 0)
    def _():
        m_ref[0] = -1e30
        s_ref[0] = 0.0
        acc_ref[...] = jnp.zeros_like(acc_ref)

    u = h_ref[...] * epsp_ref[...] + a0_ref[...] + a1_ref[...]
    v = jnp.maximum(jnp.dot(u, w1_ref[...],
                            preferred_element_type=jnp.float32)
                    + b1_ref[...], 0.0)
    h3 = jnp.maximum(jnp.dot(v, w2_ref[...],
                             preferred_element_type=jnp.float32)
                     + b2_ref[...], 0.0)
    rowid = i * blk + lax.broadcasted_iota(jnp.int32, (blk, 1), 0)
    valid = rowid < NN
    h3 = jnp.where(valid, h3, 0.0)
    g = jnp.sum(h3 * gw_ref[...], axis=1, keepdims=True)
    g = jnp.where(valid, g, -1e30)
    mo = m_ref[0]
    mn = jnp.maximum(mo, jnp.max(g))
    corr = jnp.exp(mo - mn)
    e = jnp.where(valid, jnp.exp(g - mn), 0.0)
    s_ref[0] = s_ref[0] * corr + jnp.sum(e)
    acc_ref[...] = acc_ref[...] * corr + jnp.sum(e * h3, axis=0,
                                                 keepdims=True)
    m_ref[0] = mn

    @pl.when(i == nblk - 1)
    def _():
        hg = acc_ref[...] / s_ref[0]
        z = jnp.maximum(jnp.dot(hg, h1w_ref[...],
                                preferred_element_type=jnp.float32)
                        + h1b_ref[...], 0.0)
        lg = jnp.dot(z, h2w_ref[...],
                     preferred_element_type=jnp.float32) + h2b_ref[...]
        o_ref[...] = jnp.broadcast_to(lg, o_ref.shape)


def _final(h, a0, a1, epsp, w1, b1, w2, b2, gw, h1w, h1b, h2w, h2b):
    grid = NP // BLK
    row_spec = pl.BlockSpec((BLK, H), lambda i: (i, 0))
    par_spec = pl.BlockSpec((H, H), lambda i: (0, 0))
    vec_spec = pl.BlockSpec((1, H), lambda i: (0, 0))
    return pl.pallas_call(
        _final_body,
        grid=(grid,),
        in_specs=[row_spec, row_spec, row_spec, vec_spec, par_spec, vec_spec,
                  par_spec, vec_spec, vec_spec, par_spec, vec_spec, par_spec,
                  vec_spec],
        out_specs=pl.BlockSpec((8, H), lambda i: (0, 0)),
        out_shape=jax.ShapeDtypeStruct((8, H), jnp.float32),
        scratch_shapes=[pltpu.SMEM((1,), jnp.float32),
                        pltpu.SMEM((1,), jnp.float32),
                        pltpu.VMEM((1, H), jnp.float32)],
    )(h, a0, a1, epsp, w1, b1, w2, b2, gw, h1w, h1b, h2w, h2b)


def kernel(x, edge_index, edge_attr, batch, params):
    f32 = jnp.float32
    A = jnp.zeros((192, 128), f32)
    col = 0
    for t in range(5):
        A = A.at[OFFS[t]:OFFS[t] + VOCABS[t], col:col + EDIMS[t]].set(
            params['node_embs'][t])
        col += EDIMS[t]
    for l, name in enumerate(('g1', 'g2', 'g3')):
        A = A.at[F_ROWS + 16 * l:F_ROWS + 16 * (l + 1),
                 100 + 8 * l:100 + 8 * (l + 1)].set(params[name]['edge_emb'])
    W = jnp.zeros((128, H), f32)
    W = W.at[0:100].set(params['proj_w'])
    for l, name in enumerate(('g1', 'g2', 'g3')):
        W = W.at[100 + 8 * l:100 + 8 * (l + 1)].set(params[name]['lin_w'])
    B = jnp.zeros((192, H), f32)
    B = B.at[0:F_ROWS].set(params['proj_b'][None, :] / 5.0)
    for l, name in enumerate(('g1', 'g2', 'g3')):
        B = B.at[F_ROWS + 16 * l:F_ROWS + 16 * (l + 1)].set(
            params[name]['lin_b'][None, :])

    FT = _fold(A, W, B)
    Fp = jnp.pad(FT, ((0, 64), (0, 0)))
    T = [FT[F_ROWS + 16 * l:F_ROWS + 16 * (l + 1)] for l in range(3)]

    xpad = jnp.pad(x, ((0, NP - NN), (0, 3)))
    ei = edge_index.reshape(-1)
    civ = _prep_kernel(ei, edge_attr)

    h, R = _embed(xpad, Fp, T[0])

    bn_scale = 1.0 / jnp.sqrt(1.0 + 1e-5)
    ones = jnp.ones((1, H), f32)
    for l, name in enumerate(('g1', 'g2', 'g3')):
        bp = params[name]
        agg = _msg_kernel(R.reshape(NP * 16, H), civ, ei)
        epsp = (1.0 + bp['eps']) * ones
        b1 = bp['b1'][None, :]
        b2 = bp['b2'][None, :]
        if l < 2:
            gs = (params[f'bn{l + 1}_g'] * bn_scale)[None, :]
            gb = params[f'bn{l + 1}_b'][None, :]
            h, R = _mlp(h, agg[0], agg[1], epsp, bp['w1'], b1, bp['w2'],
                        b2, gs, gb, T[l + 1])
        else:
            gw = params['gate_w'].reshape(1, H)
            h2w = jnp.zeros((H, H), f32).at[:, :NTASK].set(params['h2_w'])
            h2b = jnp.zeros((1, H), f32).at[0, :NTASK].set(params['h2_b'])
            out = _final(h, agg[0], agg[1], epsp, bp['w1'], b1, bp['w2'],
                         b2, gw, params['h1_w'], params['h1_b'][None, :],
                         h2w, h2b)
    return out[0:1, 0:NTASK]
